# Initial kernel scaffold; baseline (speedup 1.0000x reference)
import jax
import jax.numpy as jnp
from jax.experimental import pallas as pl


def kernel(x, x_v, x_s):
    raise NotImplementedError("write your pallas kernel here")


# speedup vs baseline: 2.2809x; 2.2809x over previous
import jax
import jax.numpy as jnp
from jax import lax
from jax.experimental import pallas as pl
from jax.experimental.pallas import tpu as pltpu
from jax.experimental.pallas import tpu_sc as plsc

_BIG = jnp.int32(0x7FFFFFFF)


def _make_topk(B, N, K):
    info = plsc.get_sparse_core_info()
    NC, NS = info.num_cores, info.num_subcores
    mesh = plsc.VectorSubcoreMesh(core_axis_name="c", subcore_axis_name="s")
    NV = N // 16
    KV = K // 16

    def body(xs_hbm, gidx_hbm, xso_hbm, kd_ref, ck_ref, ci_ref, ak_ref,
             ai_ref, hist_ref, offs_ref, sbuf_ref):
        wid = lax.axis_index("s") * NC + lax.axis_index("c")
        lane = lax.iota(jnp.int32, 16)

        def _dedup(d, valid):
            if valid is None:
                ukey = d * 16 + lane
            else:
                ukey = jnp.where(valid, d * 16 + lane, _BIG)
            (sk,) = lax.sort([ukey], dimension=0, num_keys=1)
            sbuf_ref[pl.ds(0, 16)] = jnp.full((16,), -1, jnp.int32)
            sbuf_ref[pl.ds(1, 16)] = sk
            prev = plsc.load_gather(sbuf_ref, [lane])
            nxt = plsc.load_gather(sbuf_ref, [jnp.minimum(lane + 2, 16)])
            sd = sk >> 4
            boundary = sd != (prev >> 4)
            fpos = plsc.cummax(jnp.where(boundary, lane, 0))
            rk_sorted = lane - fpos
            lastm = (lane == 15) | ((nxt >> 4) != sd)
            if valid is not None:
                lastm = lastm & (sk != _BIG)
            return sk, sd, lastm, rk_sorted

        def _hist_add(d, valid):
            _, sd, lastm, rk_sorted = _dedup(d, valid)
            h = plsc.load_gather(hist_ref, [sd], mask=lastm)
            plsc.store_scatter(hist_ref, [sd], h + rk_sorted + 1, mask=lastm)

        def _zero_hist(nb):
            def z(i, c):
                hist_ref[pl.ds(i * 16, 16)] = jnp.zeros((16,), jnp.int32)
                return c
            lax.fori_loop(0, nb // 16, z, 0)

        def _hist_pass(nb, digit_fn):
            def hp(i, c):
                kd = kd_ref[pl.ds(i * 16, 16)]
                d, valid = digit_fn(kd)
                _hist_add(d, valid)
                return c
            _zero_hist(nb)
            lax.fori_loop(0, NV, hp, 0)

        def _scan_hist(nb, t):
            def sb(i, carry):
                run, nf, g = carry
                h = hist_ref[pl.ds(i * 16, 16)]
                c = plsc.cumsum(h) + run
                le = c <= t
                nf = nf + jnp.sum(jnp.where(le, 1, 0).astype(jnp.int32))
                g = jnp.maximum(g, jnp.max(jnp.where(le, c, 0)))
                return jnp.max(c), nf, g
            z = jnp.int32(0)
            _, bstar, g = lax.fori_loop(0, nb // 16, sb, (z, z, z))
            return bstar, g

        @pl.when(wid < B)
        def _():
            r = wid
            pltpu.sync_copy(xs_hbm.at[pl.ds(r * N, N)], kd_ref)

            def tf(i, c):
                u = kd_ref[pl.ds(i * 16, 16)]
                kd_ref[pl.ds(i * 16, 16)] = ~(u ^ ((u >> 31) & _BIG))
                return c
            lax.fori_loop(0, NV, tf, 0)

            t1 = jnp.int32(K - 1)
            _hist_pass(2048, lambda kd: ((kd >> 21) + 1024, None))
            b1, g1 = _scan_hist(2048, t1)
            b1v = b1 - 1024
            t2 = t1 - g1

            _hist_pass(2048, lambda kd: ((kd >> 10) & 0x7FF,
                                         (kd >> 21) == b1v))
            b2, g2 = _scan_hist(2048, t2)
            t3 = t2 - g2
            pre22 = (b1v << 11) | b2

            _hist_pass(1024, lambda kd: (kd & 0x3FF, (kd >> 10) == pre22))
            b3, g3 = _scan_hist(1024, t3)

            T = (b1v << 21) | (b2 << 10) | b3
            need = jnp.int32(K) - (g1 + g2 + g3)

            def cb(i, carry):
                off, trun = carry
                kd = kd_ref[pl.ds(i * 16, 16)]
                strict = kd < T
                tie = kd == T
                tord = plsc.cumsum(jnp.where(tie, 1, 0).astype(jnp.int32))
                msel = strict | (tie & ((tord + trun) <= need))
                gi = r * N + i * 16 + lane
                plsc.store_compressed(ck_ref.at[pl.ds(off, 16)], kd, mask=msel)
                plsc.store_compressed(ci_ref.at[pl.ds(off, 16)], gi, mask=msel)
                off = off + jnp.sum(jnp.where(msel, 1, 0).astype(jnp.int32))
                trun = trun + jnp.max(tord)
                return off, trun
            lax.fori_loop(0, NV, cb, (jnp.int32(0), jnp.int32(0)))

            for p in range(4):
                src_k, src_i = (ck_ref, ci_ref) if p % 2 == 0 else (ak_ref, ai_ref)
                dst_k, dst_i = (ak_ref, ai_ref) if p % 2 == 0 else (ck_ref, ci_ref)
                sh = 8 * p

                def dig(kd, sh=sh, p=p):
                    d = (kd >> sh) & 255
                    if p == 3:
                        d = d ^ 128
                    return d

                _zero_hist(256)

                def hb(i, c, src_k=src_k, dig=dig):
                    kd = src_k[pl.ds(i * 16, 16)]
                    _hist_add(dig(kd), None)
                    return c
                lax.fori_loop(0, KV, hb, 0)

                def pb(i, run):
                    h = hist_ref[pl.ds(i * 16, 16)]
                    c = plsc.cumsum(h)
                    offs_ref[pl.ds(i * 16, 16)] = run + c - h
                    return run + jnp.max(c)
                lax.fori_loop(0, 16, pb, jnp.int32(0))

                def mb(i, c, src_k=src_k, src_i=src_i, dst_k=dst_k,
                       dst_i=dst_i, dig=dig):
                    kd = src_k[pl.ds(i * 16, 16)]
                    ix = src_i[pl.ds(i * 16, 16)]
                    d = dig(kd)
                    sk, sd, lastm, rk_sorted = _dedup(d, None)
                    plsc.store_scatter(sbuf_ref, [(sk & 15) + 17], rk_sorted)
                    rk = plsc.load_gather(sbuf_ref, [lane + 17])
                    base = plsc.load_gather(offs_ref, [d])
                    pos = base + rk
                    plsc.store_scatter(dst_k, [pos], kd)
                    plsc.store_scatter(dst_i, [pos], ix)
                    o = plsc.load_gather(offs_ref, [sd], mask=lastm)
                    plsc.store_scatter(offs_ref, [sd], o + rk_sorted + 1,
                                       mask=lastm)
                    return c
                lax.fori_loop(0, KV, mb, 0)

            def ob(i, c):
                kd = ck_ref[pl.ds(i * 16, 16)]
                s = ~kd
                ck_ref[pl.ds(i * 16, 16)] = s ^ ((s >> 31) & _BIG)
                return c
            lax.fori_loop(0, KV, ob, 0)
            pltpu.sync_copy(ck_ref.at[pl.ds(0, K)], xso_hbm.at[pl.ds(r * K, K)])
            pltpu.sync_copy(ci_ref.at[pl.ds(0, K)], gidx_hbm.at[pl.ds(r * K, K)])

    return pl.kernel(
        body,
        out_type=(
            jax.ShapeDtypeStruct((B * K,), jnp.int32),
            jax.ShapeDtypeStruct((B * K,), jnp.int32),
        ),
        mesh=mesh,
        scratch_types=[
            pltpu.VMEM((N,), jnp.int32),
            pltpu.VMEM((K + 16,), jnp.int32),
            pltpu.VMEM((K + 16,), jnp.int32),
            pltpu.VMEM((K + 16,), jnp.int32),
            pltpu.VMEM((K + 16,), jnp.int32),
            pltpu.VMEM((2048,), jnp.int32),
            pltpu.VMEM((256,), jnp.int32),
            pltpu.VMEM((48,), jnp.int32),
        ],
        compiler_params=pltpu.CompilerParams(needs_layout_passes=False),
    )


def _make_gather(B, N, K, D, Dv):
    info = plsc.get_sparse_core_info()
    NC, NS = info.num_cores, info.num_subcores
    NW = NC * NS
    mesh = plsc.VectorSubcoreMesh(core_axis_name="c", subcore_axis_name="s")
    per = (B * K) // NW
    C = 128

    def body(x_hbm, xv_hbm, gidx_hbm, xo_hbm, xvo_hbm, idx_ref, xbuf_ref,
             xvbuf_ref, sem):
        wid = lax.axis_index("s") * NC + lax.axis_index("c")
        base = wid * per

        def xc(c, carry):
            o = base + c * C
            pltpu.sync_copy(gidx_hbm.at[pl.ds(o, C)], idx_ref)
            pltpu.async_copy(x_hbm.at[idx_ref], xbuf_ref, sem).wait()
            pltpu.sync_copy(xbuf_ref, xo_hbm.at[pl.ds(o, C)])
            return carry
        lax.fori_loop(0, per // C, xc, 0)

        def vc(c, carry):
            o = base + c * C
            pltpu.sync_copy(gidx_hbm.at[pl.ds(o, C)], idx_ref)
            pltpu.async_copy(xv_hbm.at[idx_ref], xvbuf_ref, sem).wait()
            pltpu.sync_copy(xvbuf_ref, xvo_hbm.at[pl.ds(o, C)])
            return carry
        lax.fori_loop(0, per // C, vc, 0)

    return pl.kernel(
        body,
        out_type=(
            jax.ShapeDtypeStruct((B * K, D), jnp.float32),
            jax.ShapeDtypeStruct((B * K, Dv), jnp.float32),
        ),
        mesh=mesh,
        scratch_types=[
            pltpu.VMEM((C,), jnp.int32),
            pltpu.VMEM((C, D), jnp.float32),
            pltpu.VMEM((C, Dv), jnp.float32),
            pltpu.SemaphoreType.DMA,
        ],
        compiler_params=pltpu.CompilerParams(
            needs_layout_passes=False, use_tc_tiling_on_sc=False),
    )


def kernel(x, x_v, x_s):
    B, N, D = x.shape
    Dv = x_v.shape[2]
    K = N // 8

    xs_bits = lax.bitcast_convert_type(x_s.reshape(B * N), jnp.int32)
    gidx, xso_bits = _make_topk(B, N, K)(xs_bits)
    xs_out = lax.bitcast_convert_type(xso_bits, jnp.float32).reshape(B, K, 1)
    xo, xvo = _make_gather(B, N, K, D, Dv)(
        x.reshape(B * N, D), x_v.reshape(B * N, Dv), gidx)
    return xo.reshape(B, K, D), xvo.reshape(B, K, Dv), xs_out


# speedup vs baseline: 5.2624x; 2.3071x over previous
import jax
import jax.numpy as jnp
from jax import lax
from jax.experimental import pallas as pl
from jax.experimental.pallas import tpu as pltpu
from jax.experimental.pallas import tpu_sc as plsc

_BIG = jnp.int32(0x7FFFFFFF)


def _make_topk(B, N, K):
    info = plsc.get_sparse_core_info()
    NC, NS = info.num_cores, info.num_subcores
    mesh = plsc.VectorSubcoreMesh(core_axis_name="c", subcore_axis_name="s")
    NV = N // 16
    KV = K // 16

    def body(xs_hbm, gidx_hbm, xso_hbm, kd_ref, ck_ref, ci_ref, ak_ref,
             ai_ref, hist_ref, offs_ref, sbuf_ref):
        wid = lax.axis_index("s") * NC + lax.axis_index("c")
        lane = lax.iota(jnp.int32, 16)

        def _dedup(d, valid):
            if valid is None:
                ukey = d * 16 + lane
            else:
                ukey = jnp.where(valid, d * 16 + lane, _BIG)
            (sk,) = lax.sort([ukey], dimension=0, num_keys=1)
            sbuf_ref[pl.ds(0, 16)] = jnp.full((16,), -1, jnp.int32)
            sbuf_ref[pl.ds(1, 16)] = sk
            prev = plsc.load_gather(sbuf_ref, [lane])
            nxt = plsc.load_gather(sbuf_ref, [jnp.minimum(lane + 2, 16)])
            sd = sk >> 4
            boundary = sd != (prev >> 4)
            fpos = plsc.cummax(jnp.where(boundary, lane, 0))
            rk_sorted = lane - fpos
            lastm = (lane == 15) | ((nxt >> 4) != sd)
            if valid is not None:
                lastm = lastm & (sk != _BIG)
            return sk, sd, lastm, rk_sorted

        def _hist_add(d, valid):
            _, sd, lastm, rk_sorted = _dedup(d, valid)
            h = plsc.load_gather(hist_ref, [sd], mask=lastm)
            plsc.store_scatter(hist_ref, [sd], h + rk_sorted + 1, mask=lastm)

        def _zero_hist(nb):
            def z(i, c):
                hist_ref[pl.ds(i * 16, 16)] = jnp.zeros((16,), jnp.int32)
                return c
            lax.fori_loop(0, nb // 16, z, 0)

        def _hist_pass(nb, digit_fn):
            def hp(i, c):
                kd = kd_ref[pl.ds(i * 16, 16)]
                d, valid = digit_fn(kd)
                _hist_add(d, valid)
                return c
            _zero_hist(nb)
            lax.fori_loop(0, NV, hp, 0)

        def _scan_hist(nb, t):
            def sb(i, carry):
                run, nf, g = carry
                h = hist_ref[pl.ds(i * 16, 16)]
                c = plsc.cumsum(h) + run
                le = c <= t
                nf = nf + jnp.sum(jnp.where(le, 1, 0).astype(jnp.int32))
                g = jnp.maximum(g, jnp.max(jnp.where(le, c, 0)))
                return jnp.max(c), nf, g
            z = jnp.int32(0)
            _, bstar, g = lax.fori_loop(0, nb // 16, sb, (z, z, z))
            return bstar, g

        @pl.when(wid < B)
        def _():
            r = wid
            pltpu.sync_copy(xs_hbm.at[pl.ds(r * N, N)], kd_ref)

            def tf(i, c):
                u = kd_ref[pl.ds(i * 16, 16)]
                kd_ref[pl.ds(i * 16, 16)] = ~(u ^ ((u >> 31) & _BIG))
                return c
            lax.fori_loop(0, NV, tf, 0)

            t1 = jnp.int32(K - 1)
            _hist_pass(2048, lambda kd: ((kd >> 21) + 1024, None))
            b1, g1 = _scan_hist(2048, t1)
            b1v = b1 - 1024
            t2 = t1 - g1

            _hist_pass(2048, lambda kd: ((kd >> 10) & 0x7FF,
                                         (kd >> 21) == b1v))
            b2, g2 = _scan_hist(2048, t2)
            t3 = t2 - g2
            pre22 = (b1v << 11) | b2

            _hist_pass(1024, lambda kd: (kd & 0x3FF, (kd >> 10) == pre22))
            b3, g3 = _scan_hist(1024, t3)

            T = (b1v << 21) | (b2 << 10) | b3
            need = jnp.int32(K) - (g1 + g2 + g3)

            def cb(i, carry):
                off, trun = carry
                kd = kd_ref[pl.ds(i * 16, 16)]
                strict = kd < T
                tie = kd == T
                tord = plsc.cumsum(jnp.where(tie, 1, 0).astype(jnp.int32))
                msel = strict | (tie & ((tord + trun) <= need))
                gi = i * 16 + lane
                plsc.store_compressed(ck_ref.at[pl.ds(off, 16)], kd, mask=msel)
                plsc.store_compressed(ci_ref.at[pl.ds(off, 16)], gi, mask=msel)
                off = off + jnp.sum(jnp.where(msel, 1, 0).astype(jnp.int32))
                trun = trun + jnp.max(tord)
                return off, trun
            lax.fori_loop(0, NV, cb, (jnp.int32(0), jnp.int32(0)))

            for p in range(4):
                src_k, src_i = (ck_ref, ci_ref) if p % 2 == 0 else (ak_ref, ai_ref)
                dst_k, dst_i = (ak_ref, ai_ref) if p % 2 == 0 else (ck_ref, ci_ref)
                sh = 8 * p

                def dig(kd, sh=sh, p=p):
                    d = (kd >> sh) & 255
                    if p == 3:
                        d = d ^ 128
                    return d

                _zero_hist(256)

                def hb(i, c, src_k=src_k, dig=dig):
                    kd = src_k[pl.ds(i * 16, 16)]
                    _hist_add(dig(kd), None)
                    return c
                lax.fori_loop(0, KV, hb, 0)

                def pb(i, run):
                    h = hist_ref[pl.ds(i * 16, 16)]
                    c = plsc.cumsum(h)
                    offs_ref[pl.ds(i * 16, 16)] = run + c - h
                    return run + jnp.max(c)
                lax.fori_loop(0, 16, pb, jnp.int32(0))

                def mb(i, c, src_k=src_k, src_i=src_i, dst_k=dst_k,
                       dst_i=dst_i, dig=dig):
                    kd = src_k[pl.ds(i * 16, 16)]
                    ix = src_i[pl.ds(i * 16, 16)]
                    d = dig(kd)
                    sk, sd, lastm, rk_sorted = _dedup(d, None)
                    plsc.store_scatter(sbuf_ref, [(sk & 15) + 17], rk_sorted)
                    rk = plsc.load_gather(sbuf_ref, [lane + 17])
                    base = plsc.load_gather(offs_ref, [d])
                    pos = base + rk
                    plsc.store_scatter(dst_k, [pos], kd)
                    plsc.store_scatter(dst_i, [pos], ix)
                    o = plsc.load_gather(offs_ref, [sd], mask=lastm)
                    plsc.store_scatter(offs_ref, [sd], o + rk_sorted + 1,
                                       mask=lastm)
                    return c
                lax.fori_loop(0, KV, mb, 0)

            def ob(i, c):
                kd = ck_ref[pl.ds(i * 16, 16)]
                s = ~kd
                ck_ref[pl.ds(i * 16, 16)] = s ^ ((s >> 31) & _BIG)
                return c
            lax.fori_loop(0, KV, ob, 0)
            pltpu.sync_copy(ck_ref.at[pl.ds(0, K)], xso_hbm.at[pl.ds(r * K, K)])
            pltpu.sync_copy(ci_ref.at[pl.ds(0, K)], gidx_hbm.at[pl.ds(r * K, K)])

    return pl.kernel(
        body,
        out_type=(
            jax.ShapeDtypeStruct((B * K,), jnp.int32),
            jax.ShapeDtypeStruct((B * K,), jnp.int32),
        ),
        mesh=mesh,
        scratch_types=[
            pltpu.VMEM((N,), jnp.int32),
            pltpu.VMEM((K + 16,), jnp.int32),
            pltpu.VMEM((K + 16,), jnp.int32),
            pltpu.VMEM((K + 16,), jnp.int32),
            pltpu.VMEM((K + 16,), jnp.int32),
            pltpu.VMEM((2048,), jnp.int32),
            pltpu.VMEM((256,), jnp.int32),
            pltpu.VMEM((48,), jnp.int32),
        ],
        compiler_params=pltpu.CompilerParams(needs_layout_passes=False),
    )


def _make_gather(B, N, K, D, Dv):
    info = plsc.get_sparse_core_info()
    NC, NS = info.num_cores, info.num_subcores
    mesh = plsc.VectorSubcoreMesh(core_axis_name="c", subcore_axis_name="s")
    KV = K // 16

    def body(xT_hbm, xvT_hbm, gidx_hbm, xoT_hbm, xvoT_hbm, idx_ref, rb0, rb1,
             ob, sem0, sem1):
        wid = lax.axis_index("s") * NC + lax.axis_index("c")
        b = wid // 2
        h = wid % 2

        pltpu.sync_copy(gidx_hbm.at[pl.ds(b * K, K)], idx_ref)

        def gather_row(rbuf):
            def g(j, c):
                iv = idx_ref[pl.ds(j * 16, 16)]
                ob[pl.ds(j * 16, 16)] = plsc.load_gather(rbuf, [iv])
                return c
            lax.fori_loop(0, KV, g, 0)

        def phase(src, dst, row0, nrows):
            pltpu.async_copy(src.at[row0], rb0, sem0)

            def pair(p, c):
                r0 = row0 + 2 * p
                pltpu.async_copy(src.at[r0 + 1], rb1, sem1)
                pltpu.make_async_copy(src.at[r0], rb0, sem0).wait()
                gather_row(rb0)
                pltpu.sync_copy(ob, dst.at[r0])

                @pl.when(2 * p + 2 < nrows)
                def _():
                    pltpu.async_copy(src.at[r0 + 2], rb0, sem0)
                pltpu.make_async_copy(src.at[r0 + 1], rb1, sem1).wait()
                gather_row(rb1)
                pltpu.sync_copy(ob, dst.at[r0 + 1])
                return c
            lax.fori_loop(0, nrows // 2, pair, 0)

        phase(xT_hbm, xoT_hbm, b * D + h * (D // 2), D // 2)
        phase(xvT_hbm, xvoT_hbm, b * Dv + h * (Dv // 2), Dv // 2)

    return pl.kernel(
        body,
        out_type=(
            jax.ShapeDtypeStruct((B * D, K), jnp.float32),
            jax.ShapeDtypeStruct((B * Dv, K), jnp.float32),
        ),
        mesh=mesh,
        scratch_types=[
            pltpu.VMEM((K,), jnp.int32),
            pltpu.VMEM((N,), jnp.float32),
            pltpu.VMEM((N,), jnp.float32),
            pltpu.VMEM((K,), jnp.float32),
            pltpu.SemaphoreType.DMA,
            pltpu.SemaphoreType.DMA,
        ],
        compiler_params=pltpu.CompilerParams(needs_layout_passes=False),
    )


def kernel(x, x_v, x_s):
    B, N, D = x.shape
    Dv = x_v.shape[2]
    K = N // 8

    xs_bits = lax.bitcast_convert_type(x_s.reshape(B * N), jnp.int32)
    gidx, xso_bits = _make_topk(B, N, K)(xs_bits)
    xs_out = lax.bitcast_convert_type(xso_bits, jnp.float32).reshape(B, K, 1)
    xT = jnp.swapaxes(x, 1, 2).reshape(B * D, N)
    xvT = jnp.swapaxes(x_v, 1, 2).reshape(B * Dv, N)
    xoT, xvoT = _make_gather(B, N, K, D, Dv)(xT, xvT, gidx)
    xo = jnp.swapaxes(xoT.reshape(B, D, K), 1, 2)
    xvo = jnp.swapaxes(xvoT.reshape(B, Dv, K), 1, 2)
    return xo, xvo, xs_out


# speedup vs baseline: 6.7397x; 1.2807x over previous
import jax
import jax.numpy as jnp
from jax import lax
from jax.experimental import pallas as pl
from jax.experimental.pallas import tpu as pltpu
from jax.experimental.pallas import tpu_sc as plsc

_BIG = jnp.int32(0x7FFFFFFF)


def _make_topk(B, N, K):
    info = plsc.get_sparse_core_info()
    NC, NS = info.num_cores, info.num_subcores
    mesh = plsc.VectorSubcoreMesh(core_axis_name="c", subcore_axis_name="s")
    NV = N // 16
    KV = K // 16

    def body(xs_hbm, gidx_hbm, xso_hbm, kd_ref, bb_ref, h16_ref, ck_ref,
             ci_ref, ak_ref, ai_ref, hist_ref, offs16_ref, sbuf_ref):
        wid = lax.axis_index("s") * NC + lax.axis_index("c")
        lane = lax.iota(jnp.int32, 16)

        def _dedup(d, valid):
            if valid is None:
                ukey = d * 16 + lane
            else:
                ukey = jnp.where(valid, d * 16 + lane, _BIG)
            (sk,) = lax.sort([ukey], dimension=0, num_keys=1)
            sbuf_ref[pl.ds(0, 16)] = jnp.full((16,), -1, jnp.int32)
            sbuf_ref[pl.ds(1, 16)] = sk
            prev = plsc.load_gather(sbuf_ref, [lane])
            nxt = plsc.load_gather(sbuf_ref, [jnp.minimum(lane + 2, 16)])
            sd = sk >> 4
            boundary = sd != (prev >> 4)
            fpos = plsc.cummax(jnp.where(boundary, lane, 0))
            rk_sorted = lane - fpos
            lastm = (lane == 15) | ((nxt >> 4) != sd)
            if valid is not None:
                lastm = lastm & (sk != _BIG)
            return sk, sd, lastm, rk_sorted

        def _hist_add(d, valid):
            _, sd, lastm, rk_sorted = _dedup(d, valid)
            h = plsc.load_gather(hist_ref, [sd], mask=lastm)
            plsc.store_scatter(hist_ref, [sd], h + rk_sorted + 1, mask=lastm)

        def _zero_hist(nb):
            def z(i, c):
                hist_ref[pl.ds(i * 16, 16)] = jnp.zeros((16,), jnp.int32)
                return c
            lax.fori_loop(0, nb // 16, z, 0)

        def _scan_hist(nb, t):
            def sb(i, carry):
                run, nf, g = carry
                h = hist_ref[pl.ds(i * 16, 16)]
                c = plsc.cumsum(h) + run
                le = c <= t
                nf = nf + jnp.sum(jnp.where(le, 1, 0).astype(jnp.int32))
                g = jnp.maximum(g, jnp.max(jnp.where(le, c, 0)))
                return jnp.max(c), nf, g
            z = jnp.int32(0)
            _, bstar, g = lax.fori_loop(0, nb // 16, sb, (z, z, z))
            return bstar, g

        @pl.when(wid < B)
        def _():
            r = wid
            pltpu.sync_copy(xs_hbm.at[pl.ds(r * N, N)], kd_ref)

            def z1(i, c):
                h16_ref[pl.ds(i * 16, 16)] = jnp.zeros((16,), jnp.int32)
                return c
            lax.fori_loop(0, 2048, z1, 0)

            def p1(i, c):
                u = kd_ref[pl.ds(i * 16, 16)]
                kd = ~(u ^ ((u >> 31) & _BIG))
                kd_ref[pl.ds(i * 16, 16)] = kd
                a = lane * 2048 + ((kd >> 21) + 1024)
                g = plsc.load_gather(h16_ref, [a])
                plsc.store_scatter(h16_ref, [a], g + 1)
                return c
            lax.fori_loop(0, NV, p1, 0)

            t1 = jnp.int32(K - 1)

            def s1(j, carry):
                run, nf, g = carry
                tot = h16_ref[pl.ds(j * 16, 16)]
                for l in range(1, 16):
                    tot = tot + h16_ref[pl.ds(l * 2048 + j * 16, 16)]
                c = plsc.cumsum(tot) + run
                le = c <= t1
                nf = nf + jnp.sum(jnp.where(le, 1, 0).astype(jnp.int32))
                g = jnp.maximum(g, jnp.max(jnp.where(le, c, 0)))
                return jnp.max(c), nf, g
            z = jnp.int32(0)
            _, b1, g1 = lax.fori_loop(0, 128, s1, (z, z, z))
            b1v = b1 - 1024
            t2 = t1 - g1

            def cbb(i, off):
                kd = kd_ref[pl.ds(i * 16, 16)]
                m = (kd >> 21) == b1v
                plsc.store_compressed(bb_ref.at[pl.ds(off, 16)], kd, mask=m)
                return off + jnp.sum(jnp.where(m, 1, 0).astype(jnp.int32))
            m1 = lax.fori_loop(0, NV, cbb, jnp.int32(0))
            m1v = (m1 + 15) // 16

            _zero_hist(2048)

            def h2(i, c):
                kd = bb_ref[pl.ds(i * 16, 16)]
                _hist_add((kd >> 10) & 0x7FF, (i * 16 + lane) < m1)
                return c
            lax.fori_loop(0, m1v, h2, 0)
            b2, g2 = _scan_hist(2048, t2)
            t3 = t2 - g2

            def c2(i, off):
                kd = bb_ref[pl.ds(i * 16, 16)]
                m = ((i * 16 + lane) < m1) & (((kd >> 10) & 0x7FF) == b2)
                plsc.store_compressed(bb_ref.at[pl.ds(off, 16)], kd, mask=m)
                return off + jnp.sum(jnp.where(m, 1, 0).astype(jnp.int32))
            m2 = lax.fori_loop(0, m1v, c2, jnp.int32(0))
            m2v = (m2 + 15) // 16

            _zero_hist(1024)

            def h3(i, c):
                kd = bb_ref[pl.ds(i * 16, 16)]
                _hist_add(kd & 0x3FF, (i * 16 + lane) < m2)
                return c
            lax.fori_loop(0, m2v, h3, 0)
            b3, g3 = _scan_hist(1024, t3)

            T = (b1v << 21) | (b2 << 10) | b3
            need = jnp.int32(K) - (g1 + g2 + g3)

            def cb(i, carry):
                off, trun = carry
                kd = kd_ref[pl.ds(i * 16, 16)]
                strict = kd < T
                tie = kd == T
                tord = plsc.cumsum(jnp.where(tie, 1, 0).astype(jnp.int32))
                msel = strict | (tie & ((tord + trun) <= need))
                gi = i * 16 + lane
                plsc.store_compressed(ck_ref.at[pl.ds(off, 16)], kd, mask=msel)
                plsc.store_compressed(ci_ref.at[pl.ds(off, 16)], gi, mask=msel)
                off = off + jnp.sum(jnp.where(msel, 1, 0).astype(jnp.int32))
                trun = trun + jnp.max(tord)
                return off, trun
            lax.fori_loop(0, NV, cb, (jnp.int32(0), jnp.int32(0)))

            for p in range(4):
                src_k, src_i = (ck_ref, ci_ref) if p % 2 == 0 else (ak_ref, ai_ref)
                dst_k, dst_i = (ak_ref, ai_ref) if p % 2 == 0 else (ck_ref, ci_ref)
                sh = 8 * p

                def dig(kd, sh=sh, p=p):
                    d = (kd >> sh) & 255
                    if p == 3:
                        d = d ^ 128
                    return d

                def zr(i, c):
                    h16_ref[pl.ds(i * 16, 16)] = jnp.zeros((16,), jnp.int32)
                    return c
                lax.fori_loop(0, 256, zr, 0)

                def hb(i, c, src_k=src_k, dig=dig):
                    kd = plsc.load_gather(src_k, [lane * KV + i])
                    a = dig(kd) * 16 + lane
                    g = plsc.load_gather(h16_ref, [a])
                    plsc.store_scatter(h16_ref, [a], g + 1)
                    return c
                lax.fori_loop(0, KV, hb, 0)

                def pb(d, run):
                    h = h16_ref[pl.ds(d * 16, 16)]
                    c = plsc.cumsum(h)
                    offs16_ref[pl.ds(d * 16, 16)] = run + c - h
                    return run + jnp.max(c)
                lax.fori_loop(0, 256, pb, jnp.int32(0))

                def mb(i, c, src_k=src_k, src_i=src_i, dst_k=dst_k,
                       dst_i=dst_i, dig=dig):
                    kd = plsc.load_gather(src_k, [lane * KV + i])
                    ix = plsc.load_gather(src_i, [lane * KV + i])
                    a = dig(kd) * 16 + lane
                    pos = plsc.load_gather(offs16_ref, [a])
                    plsc.store_scatter(offs16_ref, [a], pos + 1)
                    plsc.store_scatter(dst_k, [pos], kd)
                    plsc.store_scatter(dst_i, [pos], ix)
                    return c
                lax.fori_loop(0, KV, mb, 0)

            def ob(i, c):
                kd = ck_ref[pl.ds(i * 16, 16)]
                s = ~kd
                ck_ref[pl.ds(i * 16, 16)] = s ^ ((s >> 31) & _BIG)
                return c
            lax.fori_loop(0, KV, ob, 0)
            pltpu.sync_copy(ck_ref.at[pl.ds(0, K)], xso_hbm.at[pl.ds(r * K, K)])
            pltpu.sync_copy(ci_ref.at[pl.ds(0, K)], gidx_hbm.at[pl.ds(r * K, K)])

    return pl.kernel(
        body,
        out_type=(
            jax.ShapeDtypeStruct((B * K,), jnp.int32),
            jax.ShapeDtypeStruct((B * K,), jnp.int32),
        ),
        mesh=mesh,
        scratch_types=[
            pltpu.VMEM((N,), jnp.int32),
            pltpu.VMEM((N,), jnp.int32),
            pltpu.VMEM((16 * 2048,), jnp.int32),
            pltpu.VMEM((K + 16,), jnp.int32),
            pltpu.VMEM((K + 16,), jnp.int32),
            pltpu.VMEM((K + 16,), jnp.int32),
            pltpu.VMEM((K + 16,), jnp.int32),
            pltpu.VMEM((2048,), jnp.int32),
            pltpu.VMEM((256 * 16,), jnp.int32),
            pltpu.VMEM((48,), jnp.int32),
        ],
        compiler_params=pltpu.CompilerParams(needs_layout_passes=False),
    )


def _make_gather(B, N, K, D, Dv):
    info = plsc.get_sparse_core_info()
    NC, NS = info.num_cores, info.num_subcores
    mesh = plsc.VectorSubcoreMesh(core_axis_name="c", subcore_axis_name="s")
    KV = K // 16

    def body(xT_hbm, xvT_hbm, gidx_hbm, xoT_hbm, xvoT_hbm, idx_ref, rb0, rb1,
             ob, sem0, sem1):
        wid = lax.axis_index("s") * NC + lax.axis_index("c")
        b = wid // 2
        h = wid % 2

        pltpu.sync_copy(gidx_hbm.at[pl.ds(b * K, K)], idx_ref)

        def gather_row(rbuf):
            def g(j, c):
                iv = idx_ref[pl.ds(j * 16, 16)]
                ob[pl.ds(j * 16, 16)] = plsc.load_gather(rbuf, [iv])
                return c
            lax.fori_loop(0, KV, g, 0)

        def phase(src, dst, row0, nrows):
            pltpu.async_copy(src.at[row0], rb0, sem0)

            def pair(p, c):
                r0 = row0 + 2 * p
                pltpu.async_copy(src.at[r0 + 1], rb1, sem1)
                pltpu.make_async_copy(src.at[r0], rb0, sem0).wait()
                gather_row(rb0)
                pltpu.sync_copy(ob, dst.at[r0])

                @pl.when(2 * p + 2 < nrows)
                def _():
                    pltpu.async_copy(src.at[r0 + 2], rb0, sem0)
                pltpu.make_async_copy(src.at[r0 + 1], rb1, sem1).wait()
                gather_row(rb1)
                pltpu.sync_copy(ob, dst.at[r0 + 1])
                return c
            lax.fori_loop(0, nrows // 2, pair, 0)

        phase(xT_hbm, xoT_hbm, b * D + h * (D // 2), D // 2)
        phase(xvT_hbm, xvoT_hbm, b * Dv + h * (Dv // 2), Dv // 2)

    return pl.kernel(
        body,
        out_type=(
            jax.ShapeDtypeStruct((B * D, K), jnp.float32),
            jax.ShapeDtypeStruct((B * Dv, K), jnp.float32),
        ),
        mesh=mesh,
        scratch_types=[
            pltpu.VMEM((K,), jnp.int32),
            pltpu.VMEM((N,), jnp.float32),
            pltpu.VMEM((N,), jnp.float32),
            pltpu.VMEM((K,), jnp.float32),
            pltpu.SemaphoreType.DMA,
            pltpu.SemaphoreType.DMA,
        ],
        compiler_params=pltpu.CompilerParams(needs_layout_passes=False),
    )


def kernel(x, x_v, x_s):
    B, N, D = x.shape
    Dv = x_v.shape[2]
    K = N // 8

    xs_bits = lax.bitcast_convert_type(x_s.reshape(B * N), jnp.int32)
    gidx, xso_bits = _make_topk(B, N, K)(xs_bits)
    xs_out = lax.bitcast_convert_type(xso_bits, jnp.float32).reshape(B, K, 1)
    xT = jnp.swapaxes(x, 1, 2).reshape(B * D, N)
    xvT = jnp.swapaxes(x_v, 1, 2).reshape(B * Dv, N)
    xoT, xvoT = _make_gather(B, N, K, D, Dv)(xT, xvT, gidx)
    xo = jnp.swapaxes(xoT.reshape(B, D, K), 1, 2)
    xvo = jnp.swapaxes(xvoT.reshape(B, Dv, K), 1, 2)
    return xo, xvo, xs_out


# speedup vs baseline: 7.8077x; 1.1585x over previous
import jax
import jax.numpy as jnp
from jax import lax
from jax.experimental import pallas as pl
from jax.experimental.pallas import tpu as pltpu
from jax.experimental.pallas import tpu_sc as plsc

_BIG = jnp.int32(0x7FFFFFFF)


def _make_topk(B, N, K):
    info = plsc.get_sparse_core_info()
    NC, NS = info.num_cores, info.num_subcores
    mesh = plsc.VectorSubcoreMesh(core_axis_name="c", subcore_axis_name="s")
    NV = N // 16
    KV = K // 16

    def body(xs_hbm, gidx_hbm, xso_hbm, kd_ref, bb_ref, h16_ref, ck_ref,
             ci_ref, ak_ref, ai_ref, hist_ref, offs16_ref, sbuf_ref):
        wid = lax.axis_index("s") * NC + lax.axis_index("c")
        lane = lax.iota(jnp.int32, 16)

        def _dedup(d, valid):
            if valid is None:
                ukey = d * 16 + lane
            else:
                ukey = jnp.where(valid, d * 16 + lane, _BIG)
            (sk,) = lax.sort([ukey], dimension=0, num_keys=1)
            sbuf_ref[pl.ds(0, 16)] = jnp.full((16,), -1, jnp.int32)
            sbuf_ref[pl.ds(1, 16)] = sk
            prev = plsc.load_gather(sbuf_ref, [lane])
            nxt = plsc.load_gather(sbuf_ref, [jnp.minimum(lane + 2, 16)])
            sd = sk >> 4
            boundary = sd != (prev >> 4)
            fpos = plsc.cummax(jnp.where(boundary, lane, 0))
            rk_sorted = lane - fpos
            lastm = (lane == 15) | ((nxt >> 4) != sd)
            if valid is not None:
                lastm = lastm & (sk != _BIG)
            return sk, sd, lastm, rk_sorted

        def _hist_add(d, valid):
            _, sd, lastm, rk_sorted = _dedup(d, valid)
            h = plsc.load_gather(hist_ref, [sd], mask=lastm)
            plsc.store_scatter(hist_ref, [sd], h + rk_sorted + 1, mask=lastm)

        def _zero_hist(nb):
            def z(i, c):
                hist_ref[pl.ds(i * 16, 16)] = jnp.zeros((16,), jnp.int32)
                return c
            lax.fori_loop(0, nb // 16, z, 0)

        def _scan_hist(nb, t):
            def sb(i, carry):
                run, nf, g = carry
                h = hist_ref[pl.ds(i * 16, 16)]
                c = plsc.cumsum(h) + run
                le = c <= t
                nf = nf + jnp.sum(jnp.where(le, 1, 0).astype(jnp.int32))
                g = jnp.maximum(g, jnp.max(jnp.where(le, c, 0)))
                return jnp.max(c), nf, g
            z = jnp.int32(0)
            _, bstar, g = lax.fori_loop(0, nb // 16, sb, (z, z, z))
            return bstar, g

        @pl.when(wid < B)
        def _():
            r = wid
            pltpu.sync_copy(xs_hbm.at[pl.ds(r * N, N)], kd_ref)

            def z1(i, c):
                for u in range(8):
                    h16_ref[pl.ds(i * 128 + u * 16, 16)] = (
                        jnp.zeros((16,), jnp.int32))
                return c
            lax.fori_loop(0, 256, z1, 0)

            def p1(i, c):
                for u in range(4):
                    uu = kd_ref[pl.ds(i * 64 + u * 16, 16)]
                    kd = ~(uu ^ ((uu >> 31) & _BIG))
                    kd_ref[pl.ds(i * 64 + u * 16, 16)] = kd
                    a = lane * 2048 + ((kd >> 21) + 1024)
                    g = plsc.load_gather(h16_ref, [a])
                    plsc.store_scatter(h16_ref, [a], g + 1)
                return c
            lax.fori_loop(0, NV // 4, p1, 0)

            t1 = jnp.int32(K - 1)

            def s1(j, carry):
                run, nf, g = carry
                tot = h16_ref[pl.ds(j * 16, 16)]
                for l in range(1, 16):
                    tot = tot + h16_ref[pl.ds(l * 2048 + j * 16, 16)]
                c = plsc.cumsum(tot) + run
                le = c <= t1
                nf = nf + jnp.sum(jnp.where(le, 1, 0).astype(jnp.int32))
                g = jnp.maximum(g, jnp.max(jnp.where(le, c, 0)))
                return jnp.max(c), nf, g
            z = jnp.int32(0)
            _, b1, g1 = lax.fori_loop(0, 128, s1, (z, z, z))
            b1v = b1 - 1024
            t2 = t1 - g1

            def cbb(i, off):
                for u in range(4):
                    kd = kd_ref[pl.ds(i * 64 + u * 16, 16)]
                    m = (kd >> 21) == b1v
                    plsc.store_compressed(bb_ref.at[pl.ds(off, 16)], kd,
                                          mask=m)
                    off = off + jnp.sum(jnp.where(m, 1, 0).astype(jnp.int32))
                return off
            m1 = lax.fori_loop(0, NV // 4, cbb, jnp.int32(0))
            m1v = (m1 + 15) // 16

            _zero_hist(2048)

            def h2(i, c):
                kd = bb_ref[pl.ds(i * 16, 16)]
                _hist_add((kd >> 10) & 0x7FF, (i * 16 + lane) < m1)
                return c
            lax.fori_loop(0, m1v, h2, 0)
            b2, g2 = _scan_hist(2048, t2)
            t3 = t2 - g2

            def c2(i, off):
                kd = bb_ref[pl.ds(i * 16, 16)]
                m = ((i * 16 + lane) < m1) & (((kd >> 10) & 0x7FF) == b2)
                plsc.store_compressed(bb_ref.at[pl.ds(off, 16)], kd, mask=m)
                return off + jnp.sum(jnp.where(m, 1, 0).astype(jnp.int32))
            m2 = lax.fori_loop(0, m1v, c2, jnp.int32(0))
            m2v = (m2 + 15) // 16

            _zero_hist(1024)

            def h3(i, c):
                kd = bb_ref[pl.ds(i * 16, 16)]
                _hist_add(kd & 0x3FF, (i * 16 + lane) < m2)
                return c
            lax.fori_loop(0, m2v, h3, 0)
            b3, g3 = _scan_hist(1024, t3)

            T = (b1v << 21) | (b2 << 10) | b3
            need = jnp.int32(K) - (g1 + g2 + g3)

            def cb(i, carry):
                off, trun = carry
                for u in range(4):
                    kd = kd_ref[pl.ds(i * 64 + u * 16, 16)]
                    strict = kd < T
                    tie = kd == T
                    tord = plsc.cumsum(jnp.where(tie, 1, 0).astype(jnp.int32))
                    msel = strict | (tie & ((tord + trun) <= need))
                    gi = i * 64 + u * 16 + lane
                    plsc.store_compressed(ck_ref.at[pl.ds(off, 16)], kd,
                                          mask=msel)
                    plsc.store_compressed(ci_ref.at[pl.ds(off, 16)], gi,
                                          mask=msel)
                    off = off + jnp.sum(jnp.where(msel, 1, 0).astype(jnp.int32))
                    trun = trun + jnp.max(tord)
                return off, trun
            lax.fori_loop(0, NV // 4, cb, (jnp.int32(0), jnp.int32(0)))

            for p in range(4):
                src_k, src_i = (ck_ref, ci_ref) if p % 2 == 0 else (ak_ref, ai_ref)
                dst_k, dst_i = (ak_ref, ai_ref) if p % 2 == 0 else (ck_ref, ci_ref)
                sh = 8 * p

                def dig(kd, sh=sh, p=p):
                    d = (kd >> sh) & 255
                    if p == 3:
                        d = d ^ 128
                    return d

                def zr(i, c):
                    for u in range(8):
                        h16_ref[pl.ds(i * 128 + u * 16, 16)] = (
                            jnp.zeros((16,), jnp.int32))
                    return c
                lax.fori_loop(0, 32, zr, 0)

                def hb(i, c, src_k=src_k, dig=dig):
                    for u in range(4):
                        kd = plsc.load_gather(src_k, [lane * KV + i * 4 + u])
                        a = dig(kd) * 16 + lane
                        g = plsc.load_gather(h16_ref, [a])
                        plsc.store_scatter(h16_ref, [a], g + 1)
                    return c
                lax.fori_loop(0, KV // 4, hb, 0)

                def pb(d, run):
                    h = h16_ref[pl.ds(d * 16, 16)]
                    c = plsc.cumsum(h)
                    offs16_ref[pl.ds(d * 16, 16)] = run + c - h
                    return run + jnp.max(c)
                lax.fori_loop(0, 256, pb, jnp.int32(0))

                def mb(i, c, src_k=src_k, src_i=src_i, dst_k=dst_k,
                       dst_i=dst_i, dig=dig):
                    for u in range(4):
                        kd = plsc.load_gather(src_k, [lane * KV + i * 4 + u])
                        ix = plsc.load_gather(src_i, [lane * KV + i * 4 + u])
                        a = dig(kd) * 16 + lane
                        pos = plsc.load_gather(offs16_ref, [a])
                        plsc.store_scatter(offs16_ref, [a], pos + 1)
                        plsc.store_scatter(dst_k, [pos], kd)
                        plsc.store_scatter(dst_i, [pos], ix)
                    return c
                lax.fori_loop(0, KV // 4, mb, 0)

            def ob(i, c):
                for u in range(4):
                    kd = ck_ref[pl.ds(i * 64 + u * 16, 16)]
                    s = ~kd
                    ck_ref[pl.ds(i * 64 + u * 16, 16)] = s ^ ((s >> 31) & _BIG)
                return c
            lax.fori_loop(0, KV // 4, ob, 0)
            pltpu.sync_copy(ck_ref.at[pl.ds(0, K)], xso_hbm.at[pl.ds(r * K, K)])
            pltpu.sync_copy(ci_ref.at[pl.ds(0, K)], gidx_hbm.at[pl.ds(r * K, K)])

    return pl.kernel(
        body,
        out_type=(
            jax.ShapeDtypeStruct((B * K,), jnp.int32),
            jax.ShapeDtypeStruct((B * K,), jnp.int32),
        ),
        mesh=mesh,
        scratch_types=[
            pltpu.VMEM((N,), jnp.int32),
            pltpu.VMEM((N,), jnp.int32),
            pltpu.VMEM((16 * 2048,), jnp.int32),
            pltpu.VMEM((K + 16,), jnp.int32),
            pltpu.VMEM((K + 16,), jnp.int32),
            pltpu.VMEM((K + 16,), jnp.int32),
            pltpu.VMEM((K + 16,), jnp.int32),
            pltpu.VMEM((2048,), jnp.int32),
            pltpu.VMEM((256 * 16,), jnp.int32),
            pltpu.VMEM((48,), jnp.int32),
        ],
        compiler_params=pltpu.CompilerParams(needs_layout_passes=False),
    )


def _make_gather(B, N, K, D, Dv):
    info = plsc.get_sparse_core_info()
    NC, NS = info.num_cores, info.num_subcores
    mesh = plsc.VectorSubcoreMesh(core_axis_name="c", subcore_axis_name="s")
    KV = K // 16

    def body(xT_hbm, xvT_hbm, gidx_hbm, xoT_hbm, xvoT_hbm, idx_ref, rb0, rb1,
             rb2, ob, sem0, sem1, sem2):
        wid = lax.axis_index("s") * NC + lax.axis_index("c")
        b = wid // 2
        h = wid % 2

        pltpu.sync_copy(gidx_hbm.at[pl.ds(b * K, K)], idx_ref)

        def gather_row(rbuf):
            def g(j, c):
                for u in range(4):
                    iv = idx_ref[pl.ds(j * 64 + u * 16, 16)]
                    ob[pl.ds(j * 64 + u * 16, 16)] = (
                        plsc.load_gather(rbuf, [iv]))
                return c
            lax.fori_loop(0, KV // 4, g, 0)

        rbs = (rb0, rb1, rb2)
        sems = (sem0, sem1, sem2)

        def phase(src, dst, row0, nrows):
            pltpu.async_copy(src.at[row0], rb0, sem0)
            pltpu.async_copy(src.at[row0 + 1], rb1, sem1)

            def trip(p, c):
                r0 = row0 + 3 * p
                for u in range(3):
                    nxt = r0 + u + 2
                    @pl.when(nxt < row0 + nrows)
                    def _():
                        pltpu.async_copy(src.at[nxt], rbs[(u + 2) % 3],
                                         sems[(u + 2) % 3])
                    pltpu.make_async_copy(src.at[r0 + u], rbs[u],
                                          sems[u]).wait()
                    gather_row(rbs[u])
                    pltpu.sync_copy(ob, dst.at[r0 + u])
                return c
            lax.fori_loop(0, nrows // 3, trip, 0)
            rest = nrows - (nrows // 3) * 3
            for u in range(rest):
                r = row0 + (nrows // 3) * 3 + u
                pltpu.make_async_copy(src.at[r], rbs[u], sems[u]).wait()
                gather_row(rbs[u])
                pltpu.sync_copy(ob, dst.at[r])

        phase(xT_hbm, xoT_hbm, b * D + h * (D // 2), D // 2)
        phase(xvT_hbm, xvoT_hbm, b * Dv + h * (Dv // 2), Dv // 2)

    return pl.kernel(
        body,
        out_type=(
            jax.ShapeDtypeStruct((B * D, K), jnp.float32),
            jax.ShapeDtypeStruct((B * Dv, K), jnp.float32),
        ),
        mesh=mesh,
        scratch_types=[
            pltpu.VMEM((K,), jnp.int32),
            pltpu.VMEM((N,), jnp.float32),
            pltpu.VMEM((N,), jnp.float32),
            pltpu.VMEM((N,), jnp.float32),
            pltpu.VMEM((K,), jnp.float32),
            pltpu.SemaphoreType.DMA,
            pltpu.SemaphoreType.DMA,
            pltpu.SemaphoreType.DMA,
        ],
        compiler_params=pltpu.CompilerParams(needs_layout_passes=False),
    )


def kernel(x, x_v, x_s):
    B, N, D = x.shape
    Dv = x_v.shape[2]
    K = N // 8

    xs_bits = lax.bitcast_convert_type(x_s.reshape(B * N), jnp.int32)
    gidx, xso_bits = _make_topk(B, N, K)(xs_bits)
    xs_out = lax.bitcast_convert_type(xso_bits, jnp.float32).reshape(B, K, 1)
    xT = jnp.swapaxes(x, 1, 2).reshape(B * D, N)
    xvT = jnp.swapaxes(x_v, 1, 2).reshape(B * Dv, N)
    xoT, xvoT = _make_gather(B, N, K, D, Dv)(xT, xvT, gidx)
    xo = jnp.swapaxes(xoT.reshape(B, D, K), 1, 2)
    xvo = jnp.swapaxes(xvoT.reshape(B, Dv, K), 1, 2)
    return xo, xvo, xs_out
